# baseline (speedup 1.0000x reference)
import functools

import jax
import jax.numpy as jnp
from jax import lax
from jax.experimental import pallas as pl
from jax.experimental.pallas import tpu as pltpu
from jax.experimental.pallas import tpu_sc as plsc

_NC = 2
_NS = 16
_NW = _NC * _NS
_CH = 80


def _fill_rows(ref, nrows, ncols, value):
    vec = jnp.full((16,), value, jnp.float32)

    def body(i, c):
        for g in range(ncols // 16):
            ref[i, 16 * g:16 * (g + 1)] = vec
        return c

    lax.fori_loop(0, nrows, body, 0)


def _sweep_chunks(n, dch, sid, fn):
    nck = n // dch
    npt = (nck + _NS - 1) // _NS

    def body(q, c):
        ck = sid + q * _NS

        @pl.when(ck < nck)
        def _():
            fn(ck * dch)

        return c

    lax.fori_loop(0, npt, body, 0)


def _sc_deg(dst1d, n):
    e = dst1d.shape[0]
    epw = e // _NW
    nit = epw // _CH
    mesh = plsc.VectorSubcoreMesh(core_axis_name="c", subcore_axis_name="s")

    @functools.partial(
        pl.kernel,
        out_type=jax.ShapeDtypeStruct((_NC, n, 16), jnp.float32),
        mesh=mesh,
        compiler_params=pltpu.CompilerParams(use_tc_tiling_on_sc=False),
        scratch_types=[
            pltpu.VMEM((e // _NW,), jnp.int32),
            [pltpu.VMEM((_CH,), jnp.int32) for _ in range(2)],
            pltpu.VMEM((_CH, 16), jnp.float32),
            pltpu.VMEM((200, 16), jnp.float32),
            pltpu.VMEM_SHARED((n, 16), jnp.float32),
            [pltpu.SemaphoreType.DMA for _ in range(2)],
        ],
    )
    def k(dst_hbm, out_hbm, flat_v, idxb, ones_v, buf_v, acc_sh, semsc):
        cid = lax.axis_index("c")
        sid = lax.axis_index("s")
        wid = cid * _NS + sid
        _fill_rows(ones_v, _CH, 16, 1.0)
        _fill_rows(buf_v, 200, 16, 0.0)
        _sweep_chunks(n, 200, sid,
                      lambda r0: pltpu.sync_copy(
                          buf_v, acc_sh.at[pl.ds(r0, 200), :]))
        pltpu.sync_copy(dst_hbm.at[pl.ds(wid * epw, epw)], flat_v)
        plsc.subcore_barrier()

        def fill_idx(j, s):
            for c2 in range(_CH // 16):
                idxb[s][16 * c2:16 * (c2 + 1)] = flat_v[
                    pl.ds(j * _CH + 16 * c2, 16)]

        def wait_scatter(s):
            pltpu.make_async_copy(ones_v, acc_sh.at[idxb[s]],
                                  semsc[s]).wait()

        def scat(s):
            pltpu.async_copy(ones_v, acc_sh.at[idxb[s]], semsc[s], add=True)

        fill_idx(0, 0)
        scat(0)

        def body(q, c):
            j0 = 2 * q
            fill_idx(j0 + 1, 1)
            scat(1)
            wait_scatter(0)

            @pl.when(j0 + 2 < nit)
            def _():
                fill_idx(j0 + 2, 0)
                scat(0)

            wait_scatter(1)
            return c

        lax.fori_loop(0, nit // 2, body, 0)
        if nit % 2:
            wait_scatter(0)
        plsc.subcore_barrier()

        def drain(r0):
            pltpu.sync_copy(acc_sh.at[pl.ds(r0, 200), :],
                            out_hbm.at[cid, pl.ds(r0, 200), :])

        _sweep_chunks(n, 200, sid, drain)

    return k(dst1d)


def _sc_agg(xs, src1d, dst1d):
    n, d = xs.shape
    e = src1d.shape[0]
    epw = e // _NW
    nit = epw // _CH
    mesh = plsc.VectorSubcoreMesh(core_axis_name="c", subcore_axis_name="s")

    @functools.partial(
        pl.kernel,
        out_type=jax.ShapeDtypeStruct((_NC, n, d), jnp.float32),
        mesh=mesh,
        scratch_types=[
            pltpu.VMEM((e // _NW,), jnp.int32),
            [pltpu.VMEM((_CH,), jnp.int32) for _ in range(3)],
            [pltpu.VMEM((_CH, d), jnp.float32) for _ in range(3)],
            pltpu.VMEM_SHARED((n, d), jnp.float32),
            [pltpu.SemaphoreType.DMA for _ in range(3)],
            [pltpu.SemaphoreType.DMA for _ in range(3)],
            [pltpu.SemaphoreType.DMA for _ in range(3)],
        ],
    )
    def k(xs_hbm, src_hbm, dst_hbm, out_hbm, srcf_v, dstb, rows, acc_sh,
          semg, semi, semsc):
        cid = lax.axis_index("c")
        sid = lax.axis_index("s")
        wid = cid * _NS + sid
        _fill_rows(rows[0], _CH, d, 0.0)

        def zero(r0):
            pltpu.async_copy(rows[0], acc_sh.at[pl.ds(r0, _CH), :], semsc[0])

        _sweep_chunks(n, _CH, sid, zero)
        pltpu.sync_copy(src_hbm.at[pl.ds(wid * epw, epw)], srcf_v)
        _sweep_chunks(n, _CH, sid,
                      lambda r0: pltpu.make_async_copy(
                          rows[0], acc_sh.at[pl.ds(r0, _CH), :],
                          semsc[0]).wait())
        plsc.subcore_barrier()

        def load_dst(j, s):
            pltpu.async_copy(dst_hbm.at[pl.ds(wid * epw + j * _CH, _CH)],
                             dstb[s], semi[s])

        def start_gather(j, s):
            pltpu.async_copy(xs_hbm.at[srcf_v.at[pl.ds(j * _CH, _CH)]],
                             rows[s], semg[s])

        def wait_gather(j, s):
            pltpu.make_async_copy(xs_hbm.at[srcf_v.at[pl.ds(j * _CH, _CH)]],
                                  rows[s], semg[s]).wait()
            pltpu.make_async_copy(dst_hbm.at[pl.ds(wid * epw + j * _CH, _CH)],
                                  dstb[s], semi[s]).wait()

        def wait_scatter(s):
            pltpu.make_async_copy(rows[s], acc_sh.at[dstb[s]],
                                  semsc[s]).wait()

        def turn(j, s, first_round):
            p = (s + 2) % 3
            wait_gather(j, s)
            pltpu.async_copy(rows[s], acc_sh.at[dstb[s]], semsc[s], add=True)

            def refill():
                if not first_round:
                    wait_scatter(p)
                load_dst(j + 2, p)
                start_gather(j + 2, p)

            if isinstance(j, int):
                if j + 2 < nit:
                    refill()
            else:
                pl.when(j + 2 < nit)(refill)

        load_dst(0, 0)
        start_gather(0, 0)
        load_dst(1, 1)
        start_gather(1, 1)

        def body0(q, c):
            turn(3 * q, 0, False)
            turn(3 * q + 1, 1, False)
            turn(3 * q + 2, 2, False)
            return c

        turn(0, 0, True)
        turn(1, 1, False)
        turn(2, 2, False)
        lax.fori_loop(1, nit // 3, body0, 0)
        for j in range(nit - (nit % 3), nit):
            turn(j, j % 3, False)
        for s in ((nit - 1) % 3, nit % 3, (nit + 1) % 3):
            wait_scatter(s)
        plsc.subcore_barrier()

        def drain(r0):
            pltpu.sync_copy(acc_sh.at[pl.ds(r0, _CH), :],
                            out_hbm.at[cid, pl.ds(r0, _CH), :])

        _sweep_chunks(n, _CH, sid, drain)

    return k(xs, src1d, dst1d)


def _tc_prep(degp, x, bn):
    n, d = x.shape

    def body(degp_ref, x_ref, dinv_ref, xs_ref):
        deg = degp_ref[0][:, 0:1] + degp_ref[1][:, 0:1] + 1.0
        dinv = lax.rsqrt(jnp.maximum(deg, 1.0))
        dinv_ref[...] = dinv
        xs_ref[...] = x_ref[...] * dinv

    return pl.pallas_call(
        body,
        grid=(n // bn,),
        in_specs=[
            pl.BlockSpec((_NC, bn, 16), lambda i: (0, i, 0)),
            pl.BlockSpec((bn, d), lambda i: (i, 0)),
        ],
        out_specs=[
            pl.BlockSpec((bn, 1), lambda i: (i, 0)),
            pl.BlockSpec((bn, d), lambda i: (i, 0)),
        ],
        out_shape=[
            jax.ShapeDtypeStruct((n, 1), jnp.float32),
            jax.ShapeDtypeStruct((n, d), jnp.float32),
        ],
    )(degp, x)


def _tc_layer(sp, xin, dinv, w, b, g, be, h1prev, bn):
    n, d = xin.shape
    nb = n // bn
    mid = h1prev is None

    def body(*refs):
        if mid:
            (sp_ref, x_ref, dinv_ref, w_ref, b_ref, g_ref, be_ref,
             o1_ref, o2_ref, hs, s1a, s2a) = refs
        else:
            (sp_ref, x_ref, dinv_ref, w_ref, b_ref, g_ref, be_ref, hp_ref,
             o1_ref, hs, s1a, s2a) = refs
        i = pl.program_id(0)

        @pl.when(i < nb)
        def _():
            dv = dinv_ref[...]
            agg = dv * (sp_ref[0] + sp_ref[1]) + (dv * dv) * x_ref[...]
            h = jnp.dot(agg, w_ref[...], preferred_element_type=jnp.float32)
            h = h + b_ref[...]
            hs[pl.ds(i * bn, bn), :] = h
            s1 = jnp.sum(h, axis=0, keepdims=True)
            s2 = jnp.sum(h * h, axis=0, keepdims=True)

            @pl.when(i == 0)
            def _():
                s1a[...] = s1
                s2a[...] = s2

            @pl.when(i > 0)
            def _():
                s1a[...] += s1
                s2a[...] += s2

        @pl.when(i >= nb)
        def _():
            mu = s1a[...] * (1.0 / n)
            ex2 = s2a[...] * (1.0 / n)
            rstd = lax.rsqrt(jnp.maximum(ex2 - mu * mu, 0.0) + 1e-5)
            hblk = hs[pl.ds((i - nb) * bn, bn), :]
            h1 = jnp.maximum((hblk - mu) * rstd * g_ref[...] + be_ref[...],
                             0.0)
            if mid:
                o1_ref[...] = h1
                o2_ref[...] = h1 * dinv_ref[...]
            else:
                o1_ref[...] = hp_ref[...] + h1

    lo = lambda i: (jnp.where(i < nb, i, 0), 0)
    hi = lambda i: (jnp.where(i < nb, 0, i - nb), 0)
    in_specs = [
        pl.BlockSpec((_NC, bn, d), lambda i: (0, jnp.where(i < nb, i, 0), 0)),
        pl.BlockSpec((bn, d), lo),
        pl.BlockSpec((bn, 1), lambda i: (i % nb, 0) if mid else lo(i)),
        pl.BlockSpec((d, d), lambda i: (0, 0)),
        pl.BlockSpec((1, d), lambda i: (0, 0)),
        pl.BlockSpec((1, d), lambda i: (0, 0)),
        pl.BlockSpec((1, d), lambda i: (0, 0)),
    ]
    args = [sp, xin, dinv, w, b, g, be]
    if mid:
        out_specs = [pl.BlockSpec((bn, d), hi), pl.BlockSpec((bn, d), hi)]
        out_shape = [jax.ShapeDtypeStruct((n, d), jnp.float32)] * 2
    else:
        in_specs.append(pl.BlockSpec((bn, d), hi))
        args.append(h1prev)
        out_specs = pl.BlockSpec((bn, d), hi)
        out_shape = jax.ShapeDtypeStruct((n, d), jnp.float32)

    return pl.pallas_call(
        body,
        grid=(2 * nb,),
        in_specs=in_specs,
        out_specs=out_specs,
        out_shape=out_shape,
        scratch_shapes=[
            pltpu.VMEM((n, d), jnp.float32),
            pltpu.VMEM((1, d), jnp.float32),
            pltpu.VMEM((1, d), jnp.float32),
        ],
    )(*args)


def kernel(x, edge_index, W1, b1, g1, be1, W2, b2, g2, be2):
    n, d = x.shape
    e = edge_index.shape[1]
    assert _CH % 16 == 0
    assert e % (_NW * _CH) == 0 and n % 200 == 0 and n % _CH == 0
    src1d = edge_index[0]
    dst1d = edge_index[1]
    b1r, g1r, be1r = b1.reshape(1, d), g1.reshape(1, d), be1.reshape(1, d)
    b2r, g2r, be2r = b2.reshape(1, d), g2.reshape(1, d), be2.reshape(1, d)
    bn = 1000

    degp = _sc_deg(dst1d, n)
    dinv, xs1 = _tc_prep(degp, x, bn)
    s1p = _sc_agg(xs1, src1d, dst1d)
    h1, xs2 = _tc_layer(s1p, x, dinv, W1, b1r, g1r, be1r, None, bn)
    s2p = _sc_agg(xs2, src1d, dst1d)
    return _tc_layer(s2p, h1, dinv, W2, b2r, g2r, be2r, h1, bn)
